# baseline (speedup 1.0000x reference)
import functools

import jax
import jax.numpy as jnp
from jax import lax
from jax.experimental import pallas as pl
from jax.experimental.pallas import tpu as pltpu
from jax.experimental.pallas import tpu_sc as plsc

NC = 2
NS = 16
CK = 128


def _pre_body(x_ref, wt_ref, b_ref, a_ref, ab_ref, h_ref, s_ref):
    h = jnp.dot(x_ref[...], wt_ref[...], preferred_element_type=jnp.float32)
    h = h + b_ref[...]
    h_ref[...] = h
    s_ref[...] = jnp.dot(h, a_ref[...], preferred_element_type=jnp.float32) + ab_ref[...]


def _tc_pre(hidden, wt, b_row, a_pad, ab_row):
    n, din = hidden.shape
    dout = wt.shape[1]
    blk = 1000
    grid = n // blk
    return pl.pallas_call(
        _pre_body,
        grid=(grid,),
        in_specs=[
            pl.BlockSpec((blk, din), lambda i: (i, 0)),
            pl.BlockSpec((din, dout), lambda i: (0, 0)),
            pl.BlockSpec((1, dout), lambda i: (0, 0)),
            pl.BlockSpec((dout, dout), lambda i: (0, 0)),
            pl.BlockSpec((1, dout), lambda i: (0, 0)),
        ],
        out_specs=[
            pl.BlockSpec((blk, dout), lambda i: (i, 0)),
            pl.BlockSpec((blk, dout), lambda i: (i, 0)),
        ],
        out_shape=[
            jax.ShapeDtypeStruct((n, dout), jnp.float32),
            jax.ShapeDtypeStruct((n, dout), jnp.float32),
        ],
    )(hidden, wt, b_row, a_pad, ab_row)


def _post_body(p0_ref, p1_ref, d0_ref, d1_ref, o_ref):
    d0 = d0_ref[...]
    d1 = d1_ref[...]
    o0 = jnp.where(d0 > 0.0, p0_ref[...] / jnp.where(d0 > 0.0, d0, 1.0), 0.0)
    o1 = jnp.where(d1 > 0.0, p1_ref[...] / jnp.where(d1 > 0.0, d1, 1.0), 0.0)
    o_ref[...] = jnp.concatenate([o0, o1], axis=1)


def _tc_post(p0, p1, d0, d1):
    n, dh = p0.shape
    blk = 1000
    grid = n // blk
    return pl.pallas_call(
        _post_body,
        grid=(grid,),
        in_specs=[
            pl.BlockSpec((blk, dh), lambda i: (i, 0)),
            pl.BlockSpec((blk, dh), lambda i: (i, 0)),
            pl.BlockSpec((blk, 1), lambda i: (i, 0)),
            pl.BlockSpec((blk, 1), lambda i: (i, 0)),
        ],
        out_specs=pl.BlockSpec((blk, 2 * dh), lambda i: (i, 0)),
        out_shape=jax.ShapeDtypeStruct((n, 2 * dh), jnp.float32),
    )(p0, p1, d0, d1)


def _build_sc(n, dh, ch, per):
    npad = ((n + NS * 128 - 1) // (NS * 128)) * (NS * 128)
    nden = npad
    rows_per_tile = npad // NS
    den_per_tile = nden // NS
    qrows = rows_per_tile // CK

    mesh = plsc.VectorSubcoreMesh(core_axis_name="c", subcore_axis_name="s")

    @functools.partial(
        pl.kernel,
        out_type=[
            jax.ShapeDtypeStruct((NC, npad, dh), jnp.float32),
            jax.ShapeDtypeStruct((NC, nden), jnp.float32),
        ],
        mesh=mesh,
        compiler_params=pltpu.CompilerParams(needs_layout_passes=False,
                                             use_tc_tiling_on_sc=False),
        scratch_types=[
            pltpu.VMEM((ch, CK), jnp.int32),
            pltpu.VMEM((ch, CK), jnp.int32),
            pltpu.VMEM((2, CK), jnp.float32),
            pltpu.VMEM((n,), jnp.float32),
            pltpu.VMEM((n,), jnp.float32),
            pltpu.VMEM((2, CK, dh), jnp.float32),
            pltpu.VMEM((den_per_tile,), jnp.float32),
            pltpu.VMEM_SHARED((npad, dh), jnp.float32),
            pltpu.VMEM_SHARED((nden,), jnp.float32),
            pltpu.SemaphoreType.DMA,
            pltpu.SemaphoreType.DMA,
            pltpu.SemaphoreType.DMA,
            pltpu.SemaphoreType.DMA,
        ],
    )
    def sc(src_hbm, dst_hbm, s1_hbm, s2_hbm, h2_hbm, outp_hbm, den_hbm,
           sidx_v, didx_v, exc_v, s1_v, s2_v, rows_v, zden_v, acc_s, den_s,
           sem_g0, sem_g1, sem_s0, sem_s1):
        cid = lax.axis_index("c")
        sid = lax.axis_index("s")
        sem_g = (sem_g0, sem_g1)
        sem_s = (sem_s0, sem_s1)
        zeros16 = jnp.zeros((16,), jnp.float32)
        lane = lax.iota(jnp.int32, 16)

        def zrow(r, _):
            for b in range(2):
                for k in range(dh // 16):
                    rows_v[b, r, pl.ds(k * 16, 16)] = zeros16
            return 0
        lax.fori_loop(0, CK, zrow, 0)

        def zden(i, _):
            zden_v[pl.ds(i * 16, 16)] = zeros16
            return 0
        lax.fori_loop(0, den_per_tile // 16, zden, 0)

        for q in range(qrows):
            pltpu.sync_copy(
                rows_v.at[0],
                acc_s.at[pl.ds(sid * rows_per_tile + q * CK, CK)])
        pltpu.sync_copy(zden_v, den_s.at[pl.ds(sid * den_per_tile, den_per_tile)])
        plsc.subcore_barrier()

        pltpu.sync_copy(src_hbm.at[sid], sidx_v)
        pltpu.sync_copy(dst_hbm.at[sid], didx_v)
        pltpu.sync_copy(s1_hbm, s1_v)
        pltpu.sync_copy(s2_hbm, s2_v)

        def compute_ex(j, b):
            for k in range(CK // 16):
                sv = sidx_v[j, pl.ds(k * 16, 16)]
                dv = didx_v[j, pl.ds(k * 16, 16)]
                e = plsc.load_gather(s1_v, [sv]) + plsc.load_gather(s2_v, [dv])
                e = jnp.where(e >= 0.0, e, e * jnp.float32(0.01))
                ex = jnp.exp(e)
                valid = (j * CK + (k * 16) + lane) < per
                exc_v[b, pl.ds(k * 16, 16)] = jnp.where(valid, ex, 0.0)

        def issue_gather(j, b):
            pltpu.async_copy(h2_hbm.at[cid].at[sidx_v.at[j]], rows_v.at[b],
                             sem_g[b])

        def wait_gather(j, b):
            pltpu.make_async_copy(h2_hbm.at[cid].at[sidx_v.at[j]],
                                  rows_v.at[b], sem_g[b]).wait()

        def scale(b):
            def grp(g, _):
                exv = exc_v[b, pl.ds(g * 16, 16)]
                for i in range(16):
                    a = exv[i]
                    r = g * 16 + i
                    for k in range(dh // 16):
                        rows_v[b, r, pl.ds(k * 16, 16)] = (
                            rows_v[b, r, pl.ds(k * 16, 16)] * a)
                return 0
            lax.fori_loop(0, CK // 16, grp, 0)

        def issue_scatter(j, b):
            pltpu.make_async_copy(rows_v.at[b], acc_s.at[didx_v.at[j]],
                                  sem_s[b]).start(add=True)
            pltpu.make_async_copy(exc_v.at[b], den_s.at[didx_v.at[j]],
                                  sem_s[b]).start(add=True)

        def wait_scatter(j, b):
            pltpu.make_async_copy(rows_v.at[b], acc_s.at[didx_v.at[j]],
                                  sem_s[b]).wait()
            pltpu.make_async_copy(exc_v.at[b], den_s.at[didx_v.at[j]],
                                  sem_s[b]).wait()

        compute_ex(0, 0)
        issue_gather(0, 0)
        compute_ex(1, 1)
        issue_gather(1, 1)
        wait_gather(0, 0)
        scale(0)
        issue_scatter(0, 0)

        def body(p, _):
            j0 = 2 * p + 2
            j1 = 2 * p + 3
            wait_scatter(j0 - 2, 0)
            compute_ex(j0, 0)
            issue_gather(j0, 0)
            wait_gather(j0 - 1, 1)
            scale(1)
            issue_scatter(j0 - 1, 1)
            wait_scatter(j1 - 2, 1)
            compute_ex(j1, 1)
            issue_gather(j1, 1)
            wait_gather(j0, 0)
            scale(0)
            issue_scatter(j0, 0)
            return 0
        lax.fori_loop(0, (ch - 2) // 2, body, 0)

        wait_gather(ch - 1, 1)
        scale(1)
        issue_scatter(ch - 1, 1)
        wait_scatter(ch - 2, 0)
        wait_scatter(ch - 1, 1)
        plsc.subcore_barrier()

        for q in range(qrows):
            b0 = sid * rows_per_tile + q * CK
            pltpu.sync_copy(acc_s.at[pl.ds(b0, CK)],
                            outp_hbm.at[cid, pl.ds(b0, CK)])
        pltpu.sync_copy(den_s.at[pl.ds(sid * den_per_tile, den_per_tile)],
                        den_hbm.at[cid, pl.ds(sid * den_per_tile, den_per_tile)])

    return sc, nden


def kernel(hidden, edge_index, W_lin, b_lin, W_att, b_att):
    n, din = hidden.shape
    dout = W_lin.shape[0]
    dh = dout // 2
    e_total = edge_index.shape[1]

    a_pad = jnp.zeros((dout, dout), jnp.float32)
    a_pad = a_pad.at[:, 0].set(W_att[0, :dout]).at[:, 1].set(W_att[0, dout:])
    ab_row = jnp.zeros((1, dout), jnp.float32).at[0, 1].set(b_att[0])
    h, s = _tc_pre(hidden, W_lin.T, b_lin.reshape(1, dout), a_pad, ab_row)
    s1 = s[:, 0]
    s2 = s[:, 1]
    h2 = jnp.stack([h[:, :dh], h[:, dh:]])

    per = e_total // NS
    ch = (per + CK - 1) // CK
    if ch % 2:
        ch += 1
    per_pad = ch * CK
    src = edge_index[0].astype(jnp.int32).reshape(NS, per)
    dst = edge_index[1].astype(jnp.int32).reshape(NS, per)
    src_p = jnp.zeros((NS, per_pad), jnp.int32).at[:, :per].set(src)
    dst_p = jnp.zeros((NS, per_pad), jnp.int32).at[:, :per].set(dst)
    src_p = src_p.reshape(NS, ch, CK)
    dst_p = dst_p.reshape(NS, ch, CK)

    sc, nden = _build_sc(n, dh, ch, per)
    outp, denp = sc(src_p, dst_p, s1, s2, h2)

    out = _tc_post(outp[0, :n], outp[1, :n],
                   denp[0, :n].reshape(n, 1), denp[1, :n].reshape(n, 1))
    return out


# speedup vs baseline: 1.2076x; 1.2076x over previous
import functools

import jax
import jax.numpy as jnp
from jax import lax
from jax.experimental import pallas as pl
from jax.experimental.pallas import tpu as pltpu
from jax.experimental.pallas import tpu_sc as plsc

NC = 2
NS = 16
NW = NC * NS
CHUNK = 64


def _pre_body(x_ref, wt_ref, b_ref, a_ref, ab_ref, h_ref, s_ref):
    h = jnp.dot(x_ref[...], wt_ref[...], preferred_element_type=jnp.float32)
    h = h + b_ref[...]
    h_ref[...] = h
    s_ref[...] = jnp.dot(h, a_ref[...], preferred_element_type=jnp.float32) + ab_ref[...]


def _tc_pre(hidden, wt, b_row, a_pad, ab_row):
    n, din = hidden.shape
    dout = wt.shape[1]
    blk = 1000
    grid = n // blk
    return pl.pallas_call(
        _pre_body,
        grid=(grid,),
        in_specs=[
            pl.BlockSpec((blk, din), lambda i: (i, 0)),
            pl.BlockSpec((din, dout), lambda i: (0, 0)),
            pl.BlockSpec((1, dout), lambda i: (0, 0)),
            pl.BlockSpec((dout, dout), lambda i: (0, 0)),
            pl.BlockSpec((1, dout), lambda i: (0, 0)),
        ],
        out_specs=[
            pl.BlockSpec((blk, dout), lambda i: (i, 0)),
            pl.BlockSpec((blk, dout), lambda i: (i, 0)),
        ],
        out_shape=[
            jax.ShapeDtypeStruct((n, dout), jnp.float32),
            jax.ShapeDtypeStruct((n, dout), jnp.float32),
        ],
    )(hidden, wt, b_row, a_pad, ab_row)


def _post_body(p0_ref, p1_ref, d0_ref, d1_ref, o_ref):
    den = d0_ref[...] + d1_ref[...]
    num = p0_ref[...] + p1_ref[...]
    o_ref[...] = jnp.where(den > 0.0, num / jnp.where(den > 0.0, den, 1.0), 0.0)


def _tc_post(p0, p1, d0, d1):
    n, d = p0.shape
    blk = 1000
    grid = n // blk
    return pl.pallas_call(
        _post_body,
        grid=(grid,),
        in_specs=[
            pl.BlockSpec((blk, d), lambda i: (i, 0)),
            pl.BlockSpec((blk, d), lambda i: (i, 0)),
            pl.BlockSpec((blk, 1), lambda i: (i, 0)),
            pl.BlockSpec((blk, 1), lambda i: (i, 0)),
        ],
        out_specs=pl.BlockSpec((blk, d), lambda i: (i, 0)),
        out_shape=jax.ShapeDtypeStruct((n, d), jnp.float32),
    )(p0, p1, d0, d1)


def _build_sc(n, d, ch, per):
    npad = ((n + NS * CHUNK - 1) // (NS * CHUNK)) * (NS * CHUNK)
    nden = npad
    rows_per_tile = npad // NS
    den_per_tile = nden // NS
    row_chunks = rows_per_tile // CHUNK
    row_step = CHUNK

    mesh = plsc.VectorSubcoreMesh(core_axis_name="c", subcore_axis_name="s")

    @functools.partial(
        pl.kernel,
        out_type=[
            jax.ShapeDtypeStruct((NC, npad, d), jnp.float32),
            jax.ShapeDtypeStruct((NC, nden), jnp.float32),
        ],
        mesh=mesh,
        compiler_params=pltpu.CompilerParams(needs_layout_passes=False),
        scratch_types=[
            pltpu.VMEM((2, CHUNK), jnp.int32),
            pltpu.VMEM((2, CHUNK), jnp.int32),
            pltpu.VMEM((2, CHUNK), jnp.float32),
            pltpu.VMEM((n,), jnp.float32),
            pltpu.VMEM((n,), jnp.float32),
            pltpu.VMEM((2, CHUNK, d), jnp.float32),
            pltpu.VMEM((nden // NS,), jnp.float32),
            pltpu.VMEM_SHARED((npad, d), jnp.float32),
            pltpu.VMEM_SHARED((nden,), jnp.float32),
            pltpu.SemaphoreType.DMA,
            pltpu.SemaphoreType.DMA,
            pltpu.SemaphoreType.DMA,
            pltpu.SemaphoreType.DMA,
        ],
    )
    def sc(src_hbm, dst_hbm, s1_hbm, s2_hbm, h_hbm, outp_hbm, den_hbm,
           sidx_v, didx_v, exc_v, s1_v, s2_v, rows_v, zden_v, acc_s, den_s,
           sem_g0, sem_g1, sem_s0, sem_s1):
        cid = lax.axis_index("c")
        sid = lax.axis_index("s")
        wid = sid * NC + cid
        sem_g = (sem_g0, sem_g1)
        sem_s = (sem_s0, sem_s1)
        zeros16 = jnp.zeros((16,), jnp.float32)
        lane = lax.iota(jnp.int32, 16)

        def zrow(r, _):
            for k in range(d // 16):
                rows_v[0, r, pl.ds(k * 16, 16)] = zeros16
            return 0
        lax.fori_loop(0, CHUNK, zrow, 0)

        def zden(i, _):
            zden_v[pl.ds(i * 16, 16)] = zeros16
            return 0
        lax.fori_loop(0, den_per_tile // 16, zden, 0)

        for c5 in range(row_chunks):
            pltpu.sync_copy(
                rows_v.at[0],
                acc_s.at[pl.ds(sid * rows_per_tile + c5 * row_step, row_step)])
        pltpu.sync_copy(zden_v, den_s.at[pl.ds(sid * den_per_tile, den_per_tile)])
        plsc.subcore_barrier()

        pltpu.sync_copy(s1_hbm, s1_v)
        pltpu.sync_copy(s2_hbm, s2_v)

        def gather_cp(b):
            return pltpu.make_async_copy(h_hbm.at[sidx_v.at[b]],
                                         rows_v.at[b], sem_g[b])

        def row_scat_cp(b):
            return pltpu.make_async_copy(rows_v.at[b],
                                         acc_s.at[didx_v.at[b]], sem_s[b])

        def den_scat_cp(b):
            return pltpu.make_async_copy(exc_v.at[b],
                                         den_s.at[didx_v.at[b]], sem_s[b])

        def compute_ex(j, b):
            for k in range(CHUNK // 16):
                sv = sidx_v[b, pl.ds(k * 16, 16)]
                dv = didx_v[b, pl.ds(k * 16, 16)]
                e = plsc.load_gather(s1_v, [sv]) + plsc.load_gather(s2_v, [dv])
                e = jnp.where(e >= 0.0, e, e * jnp.float32(0.01))
                ex = jnp.exp(e)
                valid = (j * CHUNK + (k * 16) + lane) < per
                exc_v[b, pl.ds(k * 16, 16)] = jnp.where(valid, ex, 0.0)

        def scale(b):
            def scale_grp(g, _):
                exv = exc_v[b, pl.ds(g * 16, 16)]
                for i in range(16):
                    a = exv[i]
                    r = g * 16 + i
                    for k in range(d // 16):
                        rows_v[b, r, pl.ds(k * 16, 16)] = (
                            rows_v[b, r, pl.ds(k * 16, 16)] * a)
                return 0
            lax.fori_loop(0, CHUNK // 16, scale_grp, 0)

        def fetch(j, b):
            pltpu.sync_copy(src_hbm.at[wid, j], sidx_v.at[b])
            pltpu.sync_copy(dst_hbm.at[wid, j], didx_v.at[b])
            gather_cp(b).start()
            compute_ex(j, b)

        def flush(b):
            gather_cp(b).wait()
            scale(b)
            row_scat_cp(b).start(add=True)
            den_scat_cp(b).start(add=True)

        def drain(b):
            row_scat_cp(b).wait()
            den_scat_cp(b).wait()

        fetch(0, 0)
        fetch(1, 1)
        flush(0)

        def body(p, _):
            j0 = 2 * p + 2
            j1 = 2 * p + 3
            drain(0)
            fetch(j0, 0)
            flush(1)
            drain(1)
            fetch(j1, 1)
            flush(0)
            return 0
        lax.fori_loop(0, (ch - 2) // 2, body, 0)

        flush(1)
        drain(0)
        drain(1)
        plsc.subcore_barrier()

        for c5 in range(row_chunks):
            b0 = sid * rows_per_tile + c5 * row_step
            pltpu.sync_copy(acc_s.at[pl.ds(b0, row_step)],
                            outp_hbm.at[cid, pl.ds(b0, row_step)])
        pltpu.sync_copy(den_s.at[pl.ds(sid * den_per_tile, den_per_tile)],
                        den_hbm.at[cid, pl.ds(sid * den_per_tile, den_per_tile)])

    return sc, nden


def kernel(hidden, edge_index, W_lin, b_lin, W_att, b_att):
    n, din = hidden.shape
    dout = W_lin.shape[0]
    e_total = edge_index.shape[1]

    a_pad = jnp.zeros((dout, dout), jnp.float32)
    a_pad = a_pad.at[:, 0].set(W_att[0, :dout]).at[:, 1].set(W_att[0, dout:])
    ab_row = jnp.zeros((1, dout), jnp.float32).at[0, 1].set(b_att[0])
    h, s = _tc_pre(hidden, W_lin.T, b_lin.reshape(1, dout), a_pad, ab_row)
    s1 = s[:, 0]
    s2 = s[:, 1]

    per = e_total // NW
    ch = (per + CHUNK - 1) // CHUNK
    if ch % 2:
        ch += 1
    per_pad = ch * CHUNK
    src = edge_index[0].astype(jnp.int32).reshape(NW, per)
    dst = edge_index[1].astype(jnp.int32).reshape(NW, per)
    src_p = jnp.zeros((NW, per_pad), jnp.int32).at[:, :per].set(src)
    dst_p = jnp.zeros((NW, per_pad), jnp.int32).at[:, :per].set(dst)
    src_p = src_p.reshape(NW, ch, CHUNK)
    dst_p = dst_p.reshape(NW, ch, CHUNK)

    sc, nden = _build_sc(n, dout, ch, per)
    outp, denp = sc(src_p, dst_p, s1, s2, h)

    out = _tc_post(outp[0, :n], outp[1, :n],
                   denp[0, :n].reshape(n, 1), denp[1, :n].reshape(n, 1))
    return out


# speedup vs baseline: 1.2092x; 1.0013x over previous
import functools

import jax
import jax.numpy as jnp
from jax import lax
from jax.experimental import pallas as pl
from jax.experimental.pallas import tpu as pltpu
from jax.experimental.pallas import tpu_sc as plsc

NC = 2
NS = 16
NW = NC * NS
CHUNK = 64


def _pre_body(x_ref, wt_ref, b_ref, a_ref, ab_ref, h_ref, s_ref):
    h = jnp.dot(x_ref[...], wt_ref[...], preferred_element_type=jnp.float32)
    h = h + b_ref[...]
    h_ref[...] = h
    s_ref[...] = jnp.dot(h, a_ref[...], preferred_element_type=jnp.float32) + ab_ref[...]


def _tc_pre(hidden, wt, b_row, a_pad, ab_row):
    n, din = hidden.shape
    dout = wt.shape[1]
    blk = 1000
    grid = n // blk
    return pl.pallas_call(
        _pre_body,
        grid=(grid,),
        in_specs=[
            pl.BlockSpec((blk, din), lambda i: (i, 0)),
            pl.BlockSpec((din, dout), lambda i: (0, 0)),
            pl.BlockSpec((1, dout), lambda i: (0, 0)),
            pl.BlockSpec((dout, dout), lambda i: (0, 0)),
            pl.BlockSpec((1, dout), lambda i: (0, 0)),
        ],
        out_specs=[
            pl.BlockSpec((blk, dout), lambda i: (i, 0)),
            pl.BlockSpec((blk, dout), lambda i: (i, 0)),
        ],
        out_shape=[
            jax.ShapeDtypeStruct((n, dout), jnp.float32),
            jax.ShapeDtypeStruct((n, dout), jnp.float32),
        ],
    )(hidden, wt, b_row, a_pad, ab_row)


def _post_body(p0_ref, p1_ref, d0_ref, d1_ref, o_ref):
    den = d0_ref[...] + d1_ref[...]
    num = p0_ref[...] + p1_ref[...]
    o_ref[...] = jnp.where(den > 0.0, num / jnp.where(den > 0.0, den, 1.0), 0.0)


def _tc_post(p0, p1, d0, d1):
    n, d = p0.shape
    blk = 1000
    grid = n // blk
    return pl.pallas_call(
        _post_body,
        grid=(grid,),
        in_specs=[
            pl.BlockSpec((blk, d), lambda i: (i, 0)),
            pl.BlockSpec((blk, d), lambda i: (i, 0)),
            pl.BlockSpec((blk, 1), lambda i: (i, 0)),
            pl.BlockSpec((blk, 1), lambda i: (i, 0)),
        ],
        out_specs=pl.BlockSpec((blk, d), lambda i: (i, 0)),
        out_shape=jax.ShapeDtypeStruct((n, d), jnp.float32),
    )(p0, p1, d0, d1)


def _build_sc(n, d, ch, per):
    npad = ((n + NS * CHUNK - 1) // (NS * CHUNK)) * (NS * CHUNK)
    nden = npad
    rows_per_tile = npad // NS
    den_per_tile = nden // NS
    row_chunks = rows_per_tile // CHUNK
    row_step = CHUNK

    mesh = plsc.VectorSubcoreMesh(core_axis_name="c", subcore_axis_name="s")

    @functools.partial(
        pl.kernel,
        out_type=[
            jax.ShapeDtypeStruct((NC, npad, d), jnp.float32),
            jax.ShapeDtypeStruct((NC, nden), jnp.float32),
        ],
        mesh=mesh,
        compiler_params=pltpu.CompilerParams(needs_layout_passes=False),
        scratch_types=[
            pltpu.VMEM((2, CHUNK), jnp.int32),
            pltpu.VMEM((2, CHUNK), jnp.int32),
            pltpu.VMEM((2, CHUNK), jnp.float32),
            pltpu.VMEM((n,), jnp.float32),
            pltpu.VMEM((n,), jnp.float32),
            pltpu.VMEM((2, CHUNK, d), jnp.float32),
            pltpu.VMEM((nden // NS,), jnp.float32),
            pltpu.VMEM_SHARED((npad, d), jnp.float32),
            pltpu.VMEM_SHARED((nden,), jnp.float32),
            pltpu.SemaphoreType.DMA,
            pltpu.SemaphoreType.DMA,
            pltpu.SemaphoreType.DMA,
            pltpu.SemaphoreType.DMA,
        ],
    )
    def sc(src_hbm, dst_hbm, s1_hbm, s2_hbm, h_hbm, outp_hbm, den_hbm,
           sidx_v, didx_v, exc_v, s1_v, s2_v, rows_v, zden_v, acc_s, den_s,
           sem_g0, sem_g1, sem_s0, sem_s1):
        cid = lax.axis_index("c")
        sid = lax.axis_index("s")
        wid = sid * NC + cid
        sem_g = (sem_g0, sem_g1)
        sem_s = (sem_s0, sem_s1)
        zeros16 = jnp.zeros((16,), jnp.float32)
        lane = lax.iota(jnp.int32, 16)

        def zrow(r, _):
            for k in range(d // 16):
                rows_v[0, r, pl.ds(k * 16, 16)] = zeros16
            return 0
        lax.fori_loop(0, CHUNK, zrow, 0)

        def zden(i, _):
            zden_v[pl.ds(i * 16, 16)] = zeros16
            return 0
        lax.fori_loop(0, den_per_tile // 16, zden, 0)

        for c5 in range(row_chunks):
            pltpu.sync_copy(
                rows_v.at[0],
                acc_s.at[pl.ds(sid * rows_per_tile + c5 * row_step, row_step)])
        pltpu.sync_copy(zden_v, den_s.at[pl.ds(sid * den_per_tile, den_per_tile)])
        plsc.subcore_barrier()

        pltpu.sync_copy(s1_hbm, s1_v)
        pltpu.sync_copy(s2_hbm, s2_v)

        def gather_cp(b):
            return pltpu.make_async_copy(h_hbm.at[sidx_v.at[b]],
                                         rows_v.at[b], sem_g[b])

        def row_scat_cp(b):
            return pltpu.make_async_copy(rows_v.at[b],
                                         acc_s.at[didx_v.at[b]], sem_s[b])

        def den_scat_cp(b):
            return pltpu.make_async_copy(exc_v.at[b],
                                         den_s.at[didx_v.at[b]], sem_s[b])

        def compute_ex(j, b):
            for k in range(CHUNK // 16):
                sv = sidx_v[b, pl.ds(k * 16, 16)]
                dv = didx_v[b, pl.ds(k * 16, 16)]
                e = plsc.load_gather(s1_v, [sv]) + plsc.load_gather(s2_v, [dv])
                e = jnp.where(e >= 0.0, e, e * jnp.float32(0.01))
                ex = jnp.exp(e)
                valid = (j * CHUNK + (k * 16) + lane) < per
                exc_v[b, pl.ds(k * 16, 16)] = jnp.where(valid, ex, 0.0)

        def scale(b):
            for g in range(CHUNK // 16):
                exv = exc_v[b, pl.ds(g * 16, 16)]
                for i in range(16):
                    a = exv[i]
                    r = g * 16 + i
                    for k in range(d // 16):
                        rows_v[b, r, pl.ds(k * 16, 16)] = (
                            rows_v[b, r, pl.ds(k * 16, 16)] * a)

        def fetch(j, b):
            pltpu.sync_copy(src_hbm.at[wid, j], sidx_v.at[b])
            pltpu.sync_copy(dst_hbm.at[wid, j], didx_v.at[b])
            gather_cp(b).start()
            compute_ex(j, b)

        def flush(b):
            gather_cp(b).wait()
            scale(b)
            row_scat_cp(b).start(add=True)
            den_scat_cp(b).start(add=True)

        def drain(b):
            row_scat_cp(b).wait()
            den_scat_cp(b).wait()

        fetch(0, 0)
        fetch(1, 1)
        flush(0)

        def body(p, _):
            j0 = 2 * p + 2
            j1 = 2 * p + 3
            drain(0)
            fetch(j0, 0)
            flush(1)
            drain(1)
            fetch(j1, 1)
            flush(0)
            return 0
        lax.fori_loop(0, (ch - 2) // 2, body, 0)

        flush(1)
        drain(0)
        drain(1)
        plsc.subcore_barrier()

        for c5 in range(row_chunks):
            b0 = sid * rows_per_tile + c5 * row_step
            pltpu.sync_copy(acc_s.at[pl.ds(b0, row_step)],
                            outp_hbm.at[cid, pl.ds(b0, row_step)])
        pltpu.sync_copy(den_s.at[pl.ds(sid * den_per_tile, den_per_tile)],
                        den_hbm.at[cid, pl.ds(sid * den_per_tile, den_per_tile)])

    return sc, nden


def kernel(hidden, edge_index, W_lin, b_lin, W_att, b_att):
    n, din = hidden.shape
    dout = W_lin.shape[0]
    e_total = edge_index.shape[1]

    a_pad = jnp.zeros((dout, dout), jnp.float32)
    a_pad = a_pad.at[:, 0].set(W_att[0, :dout]).at[:, 1].set(W_att[0, dout:])
    ab_row = jnp.zeros((1, dout), jnp.float32).at[0, 1].set(b_att[0])
    h, s = _tc_pre(hidden, W_lin.T, b_lin.reshape(1, dout), a_pad, ab_row)
    s1 = s[:, 0]
    s2 = s[:, 1]

    per = e_total // NW
    ch = (per + CHUNK - 1) // CHUNK
    if ch % 2:
        ch += 1
    per_pad = ch * CHUNK
    src = edge_index[0].astype(jnp.int32).reshape(NW, per)
    dst = edge_index[1].astype(jnp.int32).reshape(NW, per)
    src_p = jnp.zeros((NW, per_pad), jnp.int32).at[:, :per].set(src)
    dst_p = jnp.zeros((NW, per_pad), jnp.int32).at[:, :per].set(dst)
    src_p = src_p.reshape(NW, ch, CHUNK)
    dst_p = dst_p.reshape(NW, ch, CHUNK)

    sc, nden = _build_sc(n, dout, ch, per)
    outp, denp = sc(src_p, dst_p, s1, s2, h)

    out = _tc_post(outp[0, :n], outp[1, :n],
                   denp[0, :n].reshape(n, 1), denp[1, :n].reshape(n, 1))
    return out
